# Initial kernel scaffold; baseline (speedup 1.0000x reference)
import jax
import jax.numpy as jnp
from jax.experimental import pallas as pl


def kernel(x, mask, W_e, b_e, W1, b1, W2, b2, codebook):
    raise NotImplementedError("write your pallas kernel here")


# speedup vs baseline: 1.2394x; 1.2394x over previous
import jax
import jax.numpy as jnp
from jax.experimental import pallas as pl
from jax.experimental.pallas import tpu as pltpu

_B, _C, _H, _W = 16, 3, 384, 384
_P = 16
_PATCH_DIM = _C * _P * _P
_HID = 768
_FF = 1536
_D = 256
_K = 8192
_N_TOK = (_H // _P) * (_W // _P)
_T = _B * _N_TOK
_BT = 256
_NB = _T // _BT


def _fused_kernel(p_ref, we_ref, be_ref, w1_ref, b1_ref, w2_ref, b2_ref,
                  cbt_ref, m_ref, out_ref, c2_ref):
    @pl.when(pl.program_id(0) == 0)
    def _():
        cbt = cbt_ref[...]
        c2_ref[...] = jnp.sum(cbt * cbt, axis=0, keepdims=True)

    h = jnp.dot(p_ref[...], we_ref[...], preferred_element_type=jnp.float32) + be_ref[...]
    h = jax.nn.gelu(h)
    h = jnp.dot(h, w1_ref[...], preferred_element_type=jnp.float32) + b1_ref[...]
    h = jax.nn.gelu(h)
    h = jnp.dot(h, w2_ref[...], preferred_element_type=jnp.float32) + b2_ref[...]
    h2 = jnp.sum(h * h, axis=1, keepdims=True)
    d2 = h2 - 2.0 * jnp.dot(h, cbt_ref[...], preferred_element_type=jnp.float32) + c2_ref[...]
    dmin = jnp.min(d2, axis=1, keepdims=True)
    ks = jax.lax.broadcasted_iota(jnp.int32, d2.shape, 1)
    idx = jnp.min(jnp.where(d2 == dmin, ks, _K), axis=1).astype(jnp.int32)
    out_ref[0, 0, :] = jnp.where(m_ref[0, 0, :] != 0, idx, -1)


def kernel(x, mask, W_e, b_e, W1, b1, W2, b2, codebook):
    xp = x.reshape(_B, _C, _H // _P, _P, _W // _P, _P)
    xp = xp.transpose(0, 2, 4, 1, 3, 5).reshape(_T, _PATCH_DIM)
    cbt = codebook.T
    m = mask.reshape(_NB, 1, _BT).astype(jnp.int32)
    out = pl.pallas_call(
        _fused_kernel,
        grid=(_NB,),
        in_specs=[
            pl.BlockSpec((_BT, _PATCH_DIM), lambda i: (i, 0)),
            pl.BlockSpec((_PATCH_DIM, _HID), lambda i: (0, 0)),
            pl.BlockSpec((1, _HID), lambda i: (0, 0)),
            pl.BlockSpec((_HID, _FF), lambda i: (0, 0)),
            pl.BlockSpec((1, _FF), lambda i: (0, 0)),
            pl.BlockSpec((_FF, _D), lambda i: (0, 0)),
            pl.BlockSpec((1, _D), lambda i: (0, 0)),
            pl.BlockSpec((_D, _K), lambda i: (0, 0)),
            pl.BlockSpec((1, 1, _BT), lambda i: (i, 0, 0)),
        ],
        out_specs=pl.BlockSpec((1, 1, _BT), lambda i: (i, 0, 0)),
        out_shape=jax.ShapeDtypeStruct((_NB, 1, _BT), jnp.int32),
        scratch_shapes=[pltpu.VMEM((1, _K), jnp.float32)],
    )(xp, W_e, b_e.reshape(1, _HID), W1, b1.reshape(1, _FF), W2,
      b2.reshape(1, _D), cbt, m)
    return out.reshape(_B, _N_TOK)


# speedup vs baseline: 1.2757x; 1.0293x over previous
import jax
import jax.numpy as jnp
from jax.experimental import pallas as pl
from jax.experimental.pallas import tpu as pltpu

_B, _C, _H, _W = 16, 3, 384, 384
_P = 16
_PATCH_DIM = _C * _P * _P
_HID = 768
_FF = 1536
_D = 256
_K = 8192
_N_TOK = (_H // _P) * (_W // _P)
_T = _B * _N_TOK
_BT = 256
_NB = _T // _BT
_CK = 1024


def _fused_kernel(p_ref, we_ref, be_ref, w1_ref, b1_ref, w2_ref, b2_ref,
                  cbt_ref, m_ref, out_ref, c2_ref, hneg_ref, h2_ref):
    @pl.when(pl.program_id(0) == 0)
    def _():
        cbt = cbt_ref[...]
        c2_ref[...] = jnp.sum(cbt * cbt, axis=0, keepdims=True)

    hneg = hneg_ref[...]
    h2 = h2_ref[...]
    run_min = jnp.full((_BT, 1), jnp.inf, dtype=jnp.float32)
    run_idx = jnp.zeros((_BT, 1), dtype=jnp.float32)
    ks = jax.lax.broadcasted_iota(jnp.int32, (_BT, _CK), 1).astype(jnp.float32)
    for c in range(_K // _CK):
        a = jnp.dot(hneg, cbt_ref[:, c * _CK:(c + 1) * _CK],
                    preferred_element_type=jnp.float32)
        d2 = (h2 + a) + c2_ref[:, c * _CK:(c + 1) * _CK]
        cmin = jnp.min(d2, axis=1, keepdims=True)
        cidx = jnp.min(jnp.where(d2 == cmin, ks, jnp.inf), axis=1,
                       keepdims=True) + float(c * _CK)
        better = cmin < run_min
        run_idx = jnp.where(better, cidx, run_idx)
        run_min = jnp.where(better, cmin, run_min)
    idx = run_idx.reshape(_BT).astype(jnp.int32)
    out_ref[0, 0, :] = jnp.where(m_ref[0, 0, :] != 0, idx, -1)

    h = jnp.dot(p_ref[...], we_ref[...], preferred_element_type=jnp.float32) + be_ref[...]
    h = jax.nn.gelu(h)
    h = jnp.dot(h, w1_ref[...], preferred_element_type=jnp.float32) + b1_ref[...]
    h = jax.nn.gelu(h)
    h = jnp.dot(h, w2_ref[...], preferred_element_type=jnp.float32) + b2_ref[...]
    h2_ref[...] = jnp.sum(h * h, axis=1, keepdims=True)
    hneg_ref[...] = -2.0 * h


def kernel(x, mask, W_e, b_e, W1, b1, W2, b2, codebook):
    xp = x.reshape(_B, _C, _H // _P, _P, _W // _P, _P)
    xp = xp.transpose(0, 2, 4, 1, 3, 5).reshape(_T, _PATCH_DIM)
    cbt = codebook.T
    m = mask.reshape(_NB, 1, _BT).astype(jnp.int32)
    out = pl.pallas_call(
        _fused_kernel,
        grid=(_NB + 1,),
        in_specs=[
            pl.BlockSpec((_BT, _PATCH_DIM),
                         lambda i: (jnp.minimum(i, _NB - 1), 0)),
            pl.BlockSpec((_PATCH_DIM, _HID), lambda i: (0, 0)),
            pl.BlockSpec((1, _HID), lambda i: (0, 0)),
            pl.BlockSpec((_HID, _FF), lambda i: (0, 0)),
            pl.BlockSpec((1, _FF), lambda i: (0, 0)),
            pl.BlockSpec((_FF, _D), lambda i: (0, 0)),
            pl.BlockSpec((1, _D), lambda i: (0, 0)),
            pl.BlockSpec((_D, _K), lambda i: (0, 0)),
            pl.BlockSpec((1, 1, _BT), lambda i: (jnp.maximum(i - 1, 0), 0, 0)),
        ],
        out_specs=pl.BlockSpec((1, 1, _BT), lambda i: (jnp.maximum(i - 1, 0), 0, 0)),
        out_shape=jax.ShapeDtypeStruct((_NB, 1, _BT), jnp.int32),
        scratch_shapes=[
            pltpu.VMEM((1, _K), jnp.float32),
            pltpu.VMEM((_BT, _D), jnp.float32),
            pltpu.VMEM((_BT, 1), jnp.float32),
        ],
    )(xp, W_e, b_e.reshape(1, _HID), W1, b1.reshape(1, _FF), W2,
      b2.reshape(1, _D), cbt, m)
    return out.reshape(_B, _N_TOK)


# speedup vs baseline: 1.2977x; 1.0173x over previous
import jax
import jax.numpy as jnp
from jax.experimental import pallas as pl
from jax.experimental.pallas import tpu as pltpu

_B, _C, _H, _W = 16, 3, 384, 384
_P = 16
_PATCH_DIM = _C * _P * _P
_HID = 768
_FF = 1536
_D = 256
_K = 8192
_N_TOK = (_H // _P) * (_W // _P)
_T = _B * _N_TOK
_BT = 256
_NB = _T // _BT
_CK = 2048


def _fused_kernel(p_ref, we_ref, be_ref, w1_ref, b1_ref, w2_ref, b2_ref,
                  cb_ref, m_ref, out_ref, c2_ref, hnegt_ref, h2_ref):
    @pl.when(pl.program_id(0) == 0)
    def _():
        cb = cb_ref[...]
        c2_ref[...] = jnp.sum(cb * cb, axis=1, keepdims=True)

    hnegt = hnegt_ref[...]
    h2 = h2_ref[...]
    run_min = jnp.full((1, _BT), jnp.inf, dtype=jnp.float32)
    run_idx = jnp.zeros((1, _BT), dtype=jnp.int32)
    for c in range(_K // _CK):
        a = jnp.dot(cb_ref[c * _CK:(c + 1) * _CK, :], hnegt,
                    preferred_element_type=jnp.float32)
        d2 = (h2 + a) + c2_ref[c * _CK:(c + 1) * _CK, :]
        cmin = jnp.min(d2, axis=0, keepdims=True)
        cidx = jnp.argmin(d2, axis=0).astype(jnp.int32).reshape(1, _BT) + c * _CK
        better = cmin < run_min
        run_idx = jnp.where(better, cidx, run_idx)
        run_min = jnp.where(better, cmin, run_min)
    idx = run_idx.reshape(_BT)
    out_ref[0, 0, :] = jnp.where(m_ref[0, 0, :] != 0, idx, -1)

    h = jnp.dot(p_ref[...], we_ref[...], preferred_element_type=jnp.float32) + be_ref[...]
    h = jax.nn.gelu(h)
    h = jnp.dot(h, w1_ref[...], preferred_element_type=jnp.float32) + b1_ref[...]
    h = jax.nn.gelu(h)
    h = jnp.dot(h, w2_ref[...], preferred_element_type=jnp.float32) + b2_ref[...]
    h2_ref[...] = jnp.sum(h * h, axis=1, keepdims=True).reshape(1, _BT)
    hnegt_ref[...] = (-2.0 * h).T


def kernel(x, mask, W_e, b_e, W1, b1, W2, b2, codebook):
    xp = x.reshape(_B, _C, _H // _P, _P, _W // _P, _P)
    xp = xp.transpose(0, 2, 4, 1, 3, 5).reshape(_T, _PATCH_DIM)
    m = mask.reshape(_NB, 1, _BT).astype(jnp.int32)
    out = pl.pallas_call(
        _fused_kernel,
        grid=(_NB + 1,),
        in_specs=[
            pl.BlockSpec((_BT, _PATCH_DIM),
                         lambda i: (jnp.minimum(i, _NB - 1), 0)),
            pl.BlockSpec((_PATCH_DIM, _HID), lambda i: (0, 0)),
            pl.BlockSpec((1, _HID), lambda i: (0, 0)),
            pl.BlockSpec((_HID, _FF), lambda i: (0, 0)),
            pl.BlockSpec((1, _FF), lambda i: (0, 0)),
            pl.BlockSpec((_FF, _D), lambda i: (0, 0)),
            pl.BlockSpec((1, _D), lambda i: (0, 0)),
            pl.BlockSpec((_K, _D), lambda i: (0, 0)),
            pl.BlockSpec((1, 1, _BT), lambda i: (jnp.maximum(i - 1, 0), 0, 0)),
        ],
        out_specs=pl.BlockSpec((1, 1, _BT), lambda i: (jnp.maximum(i - 1, 0), 0, 0)),
        out_shape=jax.ShapeDtypeStruct((_NB, 1, _BT), jnp.int32),
        scratch_shapes=[
            pltpu.VMEM((_K, 1), jnp.float32),
            pltpu.VMEM((_D, _BT), jnp.float32),
            pltpu.VMEM((1, _BT), jnp.float32),
        ],
    )(xp, W_e, b_e.reshape(1, _HID), W1, b1.reshape(1, _FF), W2,
      b2.reshape(1, _D), codebook, m)
    return out.reshape(_B, _N_TOK)
